# baseline (speedup 1.0000x reference)
import functools

import jax
import jax.numpy as jnp
from jax import lax
from jax.experimental import pallas as pl
from jax.experimental.pallas import tpu as pltpu
from jax.experimental.pallas import tpu_sc as plsc

N = 10000
NG = 64
D_IN = 11
DH = 128
NPAD = 10240
PADROW = NPAD - 1
E = 320000
K = 128
NCH0 = 40
NCH1 = 120
NCHM = max(NCH0, NCH1)
NCH = NCHM
NW = 32
EPAD = NW * NCHM * K
RPT = NPAD // 16
RB = 128

_MESH = plsc.VectorSubcoreMesh(
    core_axis_name="c", subcore_axis_name="s", num_cores=2, num_subcores=16)

def _zero_acc_slice(s, zbuf, acc, F):
    zero16 = jnp.zeros((16,), jnp.float32)
    for i in range(16):
        for k in range(F // 16):
            zbuf[i, pl.ds(16 * k, 16)] = zero16

    @pl.loop(0, RPT // 16)
    def _zero(j):
        pltpu.sync_copy(zbuf, acc.at[pl.ds(s * RPT + j * 16, 16)])


def _readout(c, s, acc, stage, out_hbm):
    @pl.loop(0, RPT // RB)
    def _read(j):
        pltpu.sync_copy(acc.at[pl.ds(s * RPT + j * RB, RB)], stage)
        pltpu.sync_copy(stage, out_hbm.at[c, pl.ds(s * RPT + j * RB, RB)])


IG = 8


def _spmm_body(F, src_hbm, dst_hbm, table_hbm, out_hbm,
               sidx, didx, rows2, zbuf, acc, sem, sem_i):
    c = lax.axis_index("c")
    s = lax.axis_index("s")
    wid = c * 16 + s
    nch = jnp.where(c == 0, NCH0, NCH1)

    _zero_acc_slice(s, zbuf, acc, F)

    pltpu.async_copy(src_hbm.at[wid, pl.ds(0, IG)], sidx.at[0], sem_i)
    pltpu.async_copy(dst_hbm.at[wid, pl.ds(0, IG)], didx.at[0], sem_i)
    plsc.subcore_barrier()

    @pl.loop(0, nch + 1)
    def _edges(j):
        g = j // IG
        h = lax.rem(g, 2)
        jr = lax.rem(j, IG)

        @pl.when(j < nch)
        def _gather():
            @pl.when(jr == 0)
            def _wait_refill():
                pltpu.make_async_copy(
                    src_hbm.at[wid, pl.ds(g * IG, IG)], sidx.at[h], sem_i).wait()
                pltpu.make_async_copy(
                    dst_hbm.at[wid, pl.ds(g * IG, IG)], didx.at[h], sem_i).wait()

            pltpu.async_copy(
                table_hbm.at[sidx.at[h, jr]], rows2.at[lax.rem(j, 2)], sem)

        @pl.when(j >= 1)
        def _scatter():
            jm = j - 1
            hm = lax.rem(jm // IG, 2)
            b = lax.rem(jm, 2)
            pltpu.make_async_copy(
                table_hbm.at[sidx.at[hm, lax.rem(jm, IG)]], rows2.at[b], sem).wait()
            pltpu.sync_copy(
                rows2.at[b], acc.at[didx.at[hm, lax.rem(jm, IG)]], add=True)

        @pl.when((jr == 0) & (j + IG < nch))
        def _refill():
            h2 = lax.rem(g + 1, 2)
            pltpu.async_copy(src_hbm.at[wid, pl.ds((g + 1) * IG, IG)], sidx.at[h2], sem_i)
            pltpu.async_copy(dst_hbm.at[wid, pl.ds((g + 1) * IG, IG)], didx.at[h2], sem_i)

    plsc.subcore_barrier()
    _readout(c, s, acc, rows2.at[0], out_hbm)


def _deg_body(dst_hbm, out_hbm, didx, rows, zbuf, acc, sem):
    c = lax.axis_index("c")
    s = lax.axis_index("s")
    wid = c * 16 + s

    _zero_acc_slice(s, zbuf, acc, 16)

    one16 = jnp.ones((16,), jnp.float32)
    for i in range(RB):
        rows[i, pl.ds(0, 16)] = one16

    pltpu.sync_copy(dst_hbm.at[wid], didx)
    plsc.subcore_barrier()

    @pl.loop(0, NCHM)
    def _edges(j):
        pltpu.sync_copy(rows, acc.at[didx.at[j]], add=True)

    plsc.subcore_barrier()
    _readout(c, s, acc, rows, out_hbm)


def _make_spmm(F):
    scratch = [
        pltpu.VMEM((2, IG, K), jnp.int32),
        pltpu.VMEM((2, IG, K), jnp.int32),
        pltpu.VMEM((2, RB, F), jnp.float32),
        pltpu.VMEM((16, F), jnp.float32),
        pltpu.VMEM_SHARED((NPAD, F), jnp.float32),
        pltpu.SemaphoreType.DMA,
        pltpu.SemaphoreType.DMA,
    ]
    return pl.kernel(
        functools.partial(_spmm_body, F),
        out_type=jax.ShapeDtypeStruct((2, NPAD, F), jnp.float32),
        mesh=_MESH,
        scratch_types=scratch,
        name=f"sc_spmm_f{F}",
    )


_DEG_SCRATCH = [
    pltpu.VMEM((NCHM, K), jnp.int32),
    pltpu.VMEM((RB, 16), jnp.float32),
    pltpu.VMEM((16, 16), jnp.float32),
    pltpu.VMEM_SHARED((NPAD, 16), jnp.float32),
    pltpu.SemaphoreType.DMA,
]

_deg_kernel = pl.kernel(
    _deg_body,
    out_type=jax.ShapeDtypeStruct((2, NPAD, 16), jnp.float32),
    mesh=_MESH,
    scratch_types=_DEG_SCRATCH,
    name="sc_degree",
)

_spmm128 = _make_spmm(128)


def _scale_body(deg_ref, x_ref, xs_ref, dinv_ref):
    deg = deg_ref[0] + deg_ref[1] + 1.0
    dinv = lax.rsqrt(deg)
    rowid = lax.broadcasted_iota(jnp.int32, (NPAD, 16), 0)
    dinv = jnp.where(rowid < N, dinv, 0.0)
    xs_ref[...] = x_ref[...] * dinv[:, :1]
    dinv_ref[...] = dinv


def _layer12_body(acc_ref, xs_ref, dinv_ref, w1_ref, b1_ref, w2_ref, g2_ref):
    pre = dinv_ref[:, :1] * (acc_ref[0] + acc_ref[1] + xs_ref[...])
    h1 = jnp.dot(pre, w1_ref[...], preferred_element_type=jnp.float32) + b1_ref[...]
    h1 = jnp.maximum(h1, 0.0)
    g2 = jnp.dot(h1, w2_ref[...], preferred_element_type=jnp.float32)
    g2_ref[...] = g2 * dinv_ref[:, :1]


def _layer23_body(acc_ref, g2_ref, dinv_ref, b2_ref, w3_ref, wlin_ref, z_ref):
    d = dinv_ref[:, :1]
    h2 = d * (acc_ref[0] + acc_ref[1] + g2_ref[...]) + b2_ref[...]
    h2 = jnp.maximum(h2, 0.0)
    w3l = jnp.dot(w3_ref[...], wlin_ref[...], preferred_element_type=jnp.float32)
    z = jnp.dot(h2, w3l, preferred_element_type=jnp.float32) * d
    z_ref[...] = jnp.broadcast_to(z, (z.shape[0], DH))


def _head_body(acc_ref, z_ref, dinv_ref, batch_ref, b3_ref, wlin_ref, blin_ref, out_ref):
    c = jnp.dot(b3_ref[...], wlin_ref[...], preferred_element_type=jnp.float32)
    h3 = dinv_ref[:, :1] * (acc_ref[0][:, :1] + acc_ref[1][:, :1] + z_ref[:, :1]) + c
    onehot = (batch_ref[...] == lax.broadcasted_iota(jnp.int32, (1, NG), 1)
              ).astype(jnp.float32)
    sums = jnp.sum(onehot * h3, axis=0)
    counts = jnp.sum(onehot, axis=0)
    out_ref[...] = (sums / jnp.maximum(counts, 1.0))[:, None] + blin_ref[...]


_RB_TC = 2560


def kernel(x, edge_index, batch, W1, b1, W2, b2, W3, b3, Wlin, blin):
    f32 = jnp.float32
    e_used = 16 * (NCH0 + NCH1) * K
    epad0 = jnp.full((max(0, e_used - E),), PADROW, dtype=jnp.int32)

    def _split(idx):
        flat = jnp.concatenate([idx.astype(jnp.int32), epad0])[:e_used]
        a = flat[:16 * NCH0 * K].reshape(16, NCH0, K)
        b = flat[16 * NCH0 * K:].reshape(16, NCH1, K)
        a = jnp.pad(a, ((0, 0), (0, NCHM - NCH0), (0, 0)), constant_values=PADROW)
        b = jnp.pad(b, ((0, 0), (0, NCHM - NCH1), (0, 0)), constant_values=PADROW)
        return jnp.concatenate([a, b], axis=0)

    src_r = _split(edge_index[0])
    dst_r = _split(edge_index[1])
    x_pad = jnp.pad(x, ((0, NPAD - N), (0, DH - D_IN)))
    batch_p = jnp.pad(batch.astype(jnp.int32), (0, NPAD - N), constant_values=NG)
    batch_p = batch_p.reshape(NPAD, 1)
    W1p = jnp.pad(W1, ((0, DH - D_IN), (0, 0)))
    b1r, b2r, b3r = b1.reshape(1, DH), b2.reshape(1, DH), b3.reshape(1, DH)
    blinr = blin.reshape(1, 1)

    deg2 = _deg_kernel(dst_r)

    xs, dinv = pl.pallas_call(
        _scale_body,
        out_shape=[jax.ShapeDtypeStruct((NPAD, DH), f32),
                   jax.ShapeDtypeStruct((NPAD, 16), f32)],
    )(deg2, x_pad)

    acc1 = _spmm128(src_r, dst_r, xs)

    ng = NPAD // _RB_TC
    g2 = pl.pallas_call(
        _layer12_body,
        grid=(ng,),
        in_specs=[
            pl.BlockSpec((2, _RB_TC, DH), lambda i: (0, i, 0)),
            pl.BlockSpec((_RB_TC, DH), lambda i: (i, 0)),
            pl.BlockSpec((_RB_TC, 16), lambda i: (i, 0)),
            pl.BlockSpec((DH, DH), lambda i: (0, 0)),
            pl.BlockSpec((1, DH), lambda i: (0, 0)),
            pl.BlockSpec((DH, DH), lambda i: (0, 0)),
        ],
        out_specs=pl.BlockSpec((_RB_TC, DH), lambda i: (i, 0)),
        out_shape=jax.ShapeDtypeStruct((NPAD, DH), f32),
    )(acc1, xs, dinv, W1p, b1r, W2)

    acc2 = _spmm128(src_r, dst_r, g2)

    z = pl.pallas_call(
        _layer23_body,
        grid=(ng,),
        in_specs=[
            pl.BlockSpec((2, _RB_TC, DH), lambda i: (0, i, 0)),
            pl.BlockSpec((_RB_TC, DH), lambda i: (i, 0)),
            pl.BlockSpec((_RB_TC, 16), lambda i: (i, 0)),
            pl.BlockSpec((1, DH), lambda i: (0, 0)),
            pl.BlockSpec((DH, DH), lambda i: (0, 0)),
            pl.BlockSpec((DH, 1), lambda i: (0, 0)),
        ],
        out_specs=pl.BlockSpec((_RB_TC, DH), lambda i: (i, 0)),
        out_shape=jax.ShapeDtypeStruct((NPAD, DH), f32),
    )(acc2, g2, dinv, b2r, W3, Wlin)

    acc3 = _spmm128(src_r, dst_r, z)

    out = pl.pallas_call(
        _head_body,
        out_shape=jax.ShapeDtypeStruct((NG, 1), f32),
    )(acc3, z, dinv, batch_p, b3r, Wlin, blinr)
    return out


# speedup vs baseline: 4.0495x; 4.0495x over previous
import functools

import jax
import jax.numpy as jnp
from jax import lax
from jax.experimental import pallas as pl
from jax.experimental.pallas import tpu as pltpu
from jax.experimental.pallas import tpu_sc as plsc

N = 10000
NG = 64
D_IN = 11
DH = 128
NPAD = 10240
PADROW = NPAD - 1
E = 320000
K = 128
NCH0 = 80
NCH1 = 80
NCHM = max(NCH0, NCH1)
NCH = NCHM
NW = 32
EPAD = NW * NCHM * K
RPT = NPAD // 16
RB = 128

_MESH = plsc.VectorSubcoreMesh(
    core_axis_name="c", subcore_axis_name="s", num_cores=2, num_subcores=16)

def _zero_acc_slice(s, zbuf, acc, F):
    zero16 = jnp.zeros((16,), jnp.float32)
    for i in range(16):
        for k in range(F // 16):
            zbuf[i, pl.ds(16 * k, 16)] = zero16

    @pl.loop(0, RPT // 16)
    def _zero(j):
        pltpu.sync_copy(zbuf, acc.at[pl.ds(s * RPT + j * 16, 16)])


def _readout(c, s, acc, stage, out_hbm):
    @pl.loop(0, RPT // RB)
    def _read(j):
        pltpu.sync_copy(acc.at[pl.ds(s * RPT + j * RB, RB)], stage)
        pltpu.sync_copy(stage, out_hbm.at[c, pl.ds(s * RPT + j * RB, RB)])


IG = 8


def _spmm_body(F, src_hbm, dst_hbm, table_hbm, out_hbm,
               sidx, didx, rows2, zbuf, acc, sem, sem_i):
    c = lax.axis_index("c")
    s = lax.axis_index("s")
    wid = c * 16 + s
    nch = jnp.where(c == 0, NCH0, NCH1)

    _zero_acc_slice(s, zbuf, acc, F)

    pltpu.async_copy(src_hbm.at[wid, pl.ds(0, IG)], sidx.at[0], sem_i)
    pltpu.async_copy(dst_hbm.at[wid, pl.ds(0, IG)], didx.at[0], sem_i)
    plsc.subcore_barrier()

    @pl.loop(0, nch + 1)
    def _edges(j):
        g = j // IG
        h = lax.rem(g, 2)
        jr = lax.rem(j, IG)

        @pl.when(j < nch)
        def _gather():
            @pl.when(jr == 0)
            def _wait_refill():
                pltpu.make_async_copy(
                    src_hbm.at[wid, pl.ds(g * IG, IG)], sidx.at[h], sem_i).wait()
                pltpu.make_async_copy(
                    dst_hbm.at[wid, pl.ds(g * IG, IG)], didx.at[h], sem_i).wait()

            pltpu.async_copy(
                table_hbm.at[sidx.at[h, jr]], rows2.at[lax.rem(j, 2)], sem)

        @pl.when(j >= 1)
        def _scatter():
            jm = j - 1
            hm = lax.rem(jm // IG, 2)
            b = lax.rem(jm, 2)
            pltpu.make_async_copy(
                table_hbm.at[sidx.at[hm, lax.rem(jm, IG)]], rows2.at[b], sem).wait()
            pltpu.sync_copy(
                rows2.at[b], acc.at[didx.at[hm, lax.rem(jm, IG)]], add=True)

        @pl.when((jr == 0) & (j + IG < nch))
        def _refill():
            h2 = lax.rem(g + 1, 2)
            pltpu.async_copy(src_hbm.at[wid, pl.ds((g + 1) * IG, IG)], sidx.at[h2], sem_i)
            pltpu.async_copy(dst_hbm.at[wid, pl.ds((g + 1) * IG, IG)], didx.at[h2], sem_i)

    plsc.subcore_barrier()
    _readout(c, s, acc, rows2.at[0], out_hbm)


def _deg_body(dst_hbm, out_hbm, didx, rows, zbuf, acc, sem):
    c = lax.axis_index("c")
    s = lax.axis_index("s")
    wid = c * 16 + s

    _zero_acc_slice(s, zbuf, acc, 16)

    one16 = jnp.ones((16,), jnp.float32)
    for i in range(RB):
        rows[i, pl.ds(0, 16)] = one16

    pltpu.sync_copy(dst_hbm.at[wid], didx)
    plsc.subcore_barrier()

    @pl.loop(0, NCHM)
    def _edges(j):
        pltpu.sync_copy(rows, acc.at[didx.at[j]], add=True)

    plsc.subcore_barrier()
    _readout(c, s, acc, rows, out_hbm)


def _make_spmm(F):
    scratch = [
        pltpu.VMEM((2, IG, K), jnp.int32),
        pltpu.VMEM((2, IG, K), jnp.int32),
        pltpu.VMEM((2, RB, F), jnp.float32),
        pltpu.VMEM((16, F), jnp.float32),
        pltpu.VMEM_SHARED((NPAD, F), jnp.float32),
        pltpu.SemaphoreType.DMA,
        pltpu.SemaphoreType.DMA,
    ]
    return pl.kernel(
        functools.partial(_spmm_body, F),
        out_type=jax.ShapeDtypeStruct((2, NPAD, F), jnp.float32),
        mesh=_MESH,
        scratch_types=scratch,
        name=f"sc_spmm_f{F}",
    )


_DEG_SCRATCH = [
    pltpu.VMEM((NCHM, K), jnp.int32),
    pltpu.VMEM((RB, 16), jnp.float32),
    pltpu.VMEM((16, 16), jnp.float32),
    pltpu.VMEM_SHARED((NPAD, 16), jnp.float32),
    pltpu.SemaphoreType.DMA,
]

_deg_kernel = pl.kernel(
    _deg_body,
    out_type=jax.ShapeDtypeStruct((2, NPAD, 16), jnp.float32),
    mesh=_MESH,
    scratch_types=_DEG_SCRATCH,
    name="sc_degree",
)

_spmm128 = _make_spmm(128)


def _scale_body(deg_ref, x_ref, xs_ref, dinv_ref):
    deg = deg_ref[0] + deg_ref[1] + 1.0
    dinv = lax.rsqrt(deg)
    rowid = lax.broadcasted_iota(jnp.int32, (NPAD, 16), 0)
    dinv = jnp.where(rowid < N, dinv, 0.0)
    xs_ref[...] = x_ref[...] * dinv[:, :1]
    dinv_ref[...] = dinv


def _layer12_body(acc_ref, xs_ref, dinv_ref, w1_ref, b1_ref, w2_ref, g2_ref):
    pre = dinv_ref[:, :1] * (acc_ref[0] + acc_ref[1] + xs_ref[...])
    h1 = jnp.dot(pre, w1_ref[...], preferred_element_type=jnp.float32) + b1_ref[...]
    h1 = jnp.maximum(h1, 0.0)
    g2 = jnp.dot(h1, w2_ref[...], preferred_element_type=jnp.float32)
    g2_ref[...] = g2 * dinv_ref[:, :1]


def _layer23_body(acc_ref, g2_ref, dinv_ref, b2_ref, w3_ref, wlin_ref, z_ref):
    d = dinv_ref[:, :1]
    h2 = d * (acc_ref[0] + acc_ref[1] + g2_ref[...]) + b2_ref[...]
    h2 = jnp.maximum(h2, 0.0)
    w3l = jnp.dot(w3_ref[...], wlin_ref[...], preferred_element_type=jnp.float32)
    z = jnp.dot(h2, w3l, preferred_element_type=jnp.float32) * d
    z_ref[...] = jnp.broadcast_to(z, (z.shape[0], DH))


def _head_body(acc_ref, z_ref, dinv_ref, batch_ref, b3_ref, wlin_ref, blin_ref, out_ref):
    c = jnp.dot(b3_ref[...], wlin_ref[...], preferred_element_type=jnp.float32)
    h3 = dinv_ref[:, :1] * (acc_ref[0][:, :1] + acc_ref[1][:, :1] + z_ref[:, :1]) + c
    onehot = (batch_ref[...] == lax.broadcasted_iota(jnp.int32, (1, NG), 1)
              ).astype(jnp.float32)
    sums = jnp.sum(onehot * h3, axis=0)
    counts = jnp.sum(onehot, axis=0)
    out_ref[...] = (sums / jnp.maximum(counts, 1.0))[:, None] + blin_ref[...]


_RB_TC = 2560


def kernel(x, edge_index, batch, W1, b1, W2, b2, W3, b3, Wlin, blin):
    f32 = jnp.float32
    e_used = 16 * (NCH0 + NCH1) * K
    epad0 = N + (jnp.arange(max(0, e_used - E), dtype=jnp.int32) % (NPAD - N))

    def _split(idx):
        flat = jnp.concatenate([idx.astype(jnp.int32), epad0])[:e_used]
        a = flat[:16 * NCH0 * K].reshape(16, NCH0, K)
        b = flat[16 * NCH0 * K:].reshape(16, NCH1, K)
        a = jnp.pad(a, ((0, 0), (0, NCHM - NCH0), (0, 0)), constant_values=PADROW)
        b = jnp.pad(b, ((0, 0), (0, NCHM - NCH1), (0, 0)), constant_values=PADROW)
        return jnp.concatenate([a, b], axis=0)

    src_r = _split(edge_index[0])
    dst_r = _split(edge_index[1])
    x_pad = jnp.pad(x, ((0, NPAD - N), (0, DH - D_IN)))
    batch_p = jnp.pad(batch.astype(jnp.int32), (0, NPAD - N), constant_values=NG)
    batch_p = batch_p.reshape(NPAD, 1)
    W1p = jnp.pad(W1, ((0, DH - D_IN), (0, 0)))
    b1r, b2r, b3r = b1.reshape(1, DH), b2.reshape(1, DH), b3.reshape(1, DH)
    blinr = blin.reshape(1, 1)

    deg2 = _deg_kernel(dst_r)

    xs, dinv = pl.pallas_call(
        _scale_body,
        out_shape=[jax.ShapeDtypeStruct((NPAD, DH), f32),
                   jax.ShapeDtypeStruct((NPAD, 16), f32)],
    )(deg2, x_pad)

    acc1 = _spmm128(src_r, dst_r, xs)

    ng = NPAD // _RB_TC
    g2 = pl.pallas_call(
        _layer12_body,
        grid=(ng,),
        in_specs=[
            pl.BlockSpec((2, _RB_TC, DH), lambda i: (0, i, 0)),
            pl.BlockSpec((_RB_TC, DH), lambda i: (i, 0)),
            pl.BlockSpec((_RB_TC, 16), lambda i: (i, 0)),
            pl.BlockSpec((DH, DH), lambda i: (0, 0)),
            pl.BlockSpec((1, DH), lambda i: (0, 0)),
            pl.BlockSpec((DH, DH), lambda i: (0, 0)),
        ],
        out_specs=pl.BlockSpec((_RB_TC, DH), lambda i: (i, 0)),
        out_shape=jax.ShapeDtypeStruct((NPAD, DH), f32),
    )(acc1, xs, dinv, W1p, b1r, W2)

    acc2 = _spmm128(src_r, dst_r, g2)

    z = pl.pallas_call(
        _layer23_body,
        grid=(ng,),
        in_specs=[
            pl.BlockSpec((2, _RB_TC, DH), lambda i: (0, i, 0)),
            pl.BlockSpec((_RB_TC, DH), lambda i: (i, 0)),
            pl.BlockSpec((_RB_TC, 16), lambda i: (i, 0)),
            pl.BlockSpec((1, DH), lambda i: (0, 0)),
            pl.BlockSpec((DH, DH), lambda i: (0, 0)),
            pl.BlockSpec((DH, 1), lambda i: (0, 0)),
        ],
        out_specs=pl.BlockSpec((_RB_TC, DH), lambda i: (i, 0)),
        out_shape=jax.ShapeDtypeStruct((NPAD, DH), f32),
    )(acc2, g2, dinv, b2r, W3, Wlin)

    acc3 = _spmm128(src_r, dst_r, z)

    out = pl.pallas_call(
        _head_body,
        out_shape=jax.ShapeDtypeStruct((NG, 1), f32),
    )(acc3, z, dinv, batch_p, b3r, Wlin, blinr)
    return out


# speedup vs baseline: 5.1633x; 1.2751x over previous
import functools

import jax
import jax.numpy as jnp
from jax import lax
from jax.experimental import pallas as pl
from jax.experimental.pallas import tpu as pltpu
from jax.experimental.pallas import tpu_sc as plsc

N = 10000
NG = 64
D_IN = 11
DH = 128
NPAD = 10240
PADROW = NPAD - 1
E = 320000
K = 128
NCH0 = 80
NCH1 = 80
NCHM = max(NCH0, NCH1)
NCH = NCHM
NW = 32
EPAD = NW * NCHM * K
RPT = NPAD // 16
RB = 128

_MESH = plsc.VectorSubcoreMesh(
    core_axis_name="c", subcore_axis_name="s", num_cores=2, num_subcores=16)

def _zero_acc_slice(s, zbuf, acc, F):
    zero16 = jnp.zeros((16,), jnp.float32)
    for i in range(16):
        for k in range(F // 16):
            zbuf[i, pl.ds(16 * k, 16)] = zero16

    @pl.loop(0, RPT // 16)
    def _zero(j):
        pltpu.sync_copy(zbuf, acc.at[pl.ds(s * RPT + j * 16, 16)])


def _readout(c, s, acc, stage, out_hbm):
    @pl.loop(0, RPT // RB)
    def _read(j):
        pltpu.sync_copy(acc.at[pl.ds(s * RPT + j * RB, RB)], stage)
        pltpu.sync_copy(stage, out_hbm.at[c, pl.ds(s * RPT + j * RB, RB)])


IG = 8


def _spmm_body(F, src_hbm, dst_hbm, table_hbm, out_hbm,
               sidx, didx, rows2, zbuf, acc, sem, sem_i):
    c = lax.axis_index("c")
    s = lax.axis_index("s")
    wid = c * 16 + s
    nch = jnp.where(c == 0, NCH0, NCH1)

    _zero_acc_slice(s, zbuf, acc, F)

    pltpu.async_copy(src_hbm.at[wid, pl.ds(0, IG)], sidx.at[0], sem_i)
    pltpu.async_copy(dst_hbm.at[wid, pl.ds(0, IG)], didx.at[0], sem_i)
    plsc.subcore_barrier()

    @pl.loop(0, nch + 1)
    def _edges(j):
        g = j // IG
        h = lax.rem(g, 2)
        jr = lax.rem(j, IG)

        @pl.when(j < nch)
        def _gather():
            @pl.when(jr == 0)
            def _wait_refill():
                pltpu.make_async_copy(
                    src_hbm.at[wid, pl.ds(g * IG, IG)], sidx.at[h], sem_i).wait()
                pltpu.make_async_copy(
                    dst_hbm.at[wid, pl.ds(g * IG, IG)], didx.at[h], sem_i).wait()

            pltpu.async_copy(
                table_hbm.at[sidx.at[h, jr]], rows2.at[lax.rem(j, 2)], sem)

        @pl.when(j >= 1)
        def _scatter():
            jm = j - 1
            hm = lax.rem(jm // IG, 2)
            b = lax.rem(jm, 2)
            pltpu.make_async_copy(
                table_hbm.at[sidx.at[hm, lax.rem(jm, IG)]], rows2.at[b], sem).wait()
            pltpu.sync_copy(
                rows2.at[b], acc.at[didx.at[hm, lax.rem(jm, IG)]], add=True)

        @pl.when((jr == 0) & (j + IG < nch))
        def _refill():
            h2 = lax.rem(g + 1, 2)
            pltpu.async_copy(src_hbm.at[wid, pl.ds((g + 1) * IG, IG)], sidx.at[h2], sem_i)
            pltpu.async_copy(dst_hbm.at[wid, pl.ds((g + 1) * IG, IG)], didx.at[h2], sem_i)

    plsc.subcore_barrier()
    _readout(c, s, acc, rows2.at[0], out_hbm)


def _deg_body(dst_hbm, out_hbm, didx, rows, zbuf, acc, sem):
    c = lax.axis_index("c")
    s = lax.axis_index("s")
    wid = c * 16 + s

    _zero_acc_slice(s, zbuf, acc, 16)

    one16 = jnp.ones((16,), jnp.float32)
    for i in range(RB):
        rows[i, pl.ds(0, 16)] = one16

    pltpu.sync_copy(dst_hbm.at[wid], didx)
    plsc.subcore_barrier()

    @pl.loop(0, NCHM)
    def _edges(j):
        pltpu.sync_copy(rows, acc.at[didx.at[j]], add=True)

    plsc.subcore_barrier()
    _readout(c, s, acc, rows, out_hbm)


def _make_spmm(F, tc_tiling=True):
    scratch = [
        pltpu.VMEM((2, IG, K), jnp.int32),
        pltpu.VMEM((2, IG, K), jnp.int32),
        pltpu.VMEM((2, RB, F), jnp.float32),
        pltpu.VMEM((16, F), jnp.float32),
        pltpu.VMEM_SHARED((NPAD, F), jnp.float32),
        pltpu.SemaphoreType.DMA,
        pltpu.SemaphoreType.DMA,
    ]
    return pl.kernel(
        functools.partial(_spmm_body, F),
        out_type=jax.ShapeDtypeStruct((2, NPAD, F), jnp.float32),
        mesh=_MESH,
        scratch_types=scratch,
        compiler_params=pltpu.CompilerParams(use_tc_tiling_on_sc=tc_tiling),
        name=f"sc_spmm_f{F}",
    )


_DEG_SCRATCH = [
    pltpu.VMEM((NCHM, K), jnp.int32),
    pltpu.VMEM((RB, 16), jnp.float32),
    pltpu.VMEM((16, 16), jnp.float32),
    pltpu.VMEM_SHARED((NPAD, 16), jnp.float32),
    pltpu.SemaphoreType.DMA,
]

_deg_kernel = pl.kernel(
    _deg_body,
    out_type=jax.ShapeDtypeStruct((2, NPAD, 16), jnp.float32),
    mesh=_MESH,
    scratch_types=_DEG_SCRATCH,
    name="sc_degree",
)

_spmm128 = _make_spmm(128)
_spmm16 = _make_spmm(16, tc_tiling=False)


def _scale_body(deg_ref, x_ref, xs_ref, dinv_ref):
    deg = deg_ref[0] + deg_ref[1] + 1.0
    dinv = lax.rsqrt(deg)
    rowid = lax.broadcasted_iota(jnp.int32, (NPAD, 16), 0)
    dinv = jnp.where(rowid < N, dinv, 0.0)
    xs_ref[...] = x_ref[...] * dinv
    dinv_ref[...] = dinv


def _layer12_body(acc_ref, xs_ref, dinv_ref, w1_ref, b1_ref, w2_ref, g2_ref):
    pre = dinv_ref[...] * (acc_ref[0] + acc_ref[1] + xs_ref[...])
    h1 = jnp.dot(pre, w1_ref[...], preferred_element_type=jnp.float32) + b1_ref[...]
    h1 = jnp.maximum(h1, 0.0)
    g2 = jnp.dot(h1, w2_ref[...], preferred_element_type=jnp.float32)
    g2_ref[...] = g2 * dinv_ref[:, :1]


def _layer23_body(acc_ref, g2_ref, dinv_ref, b2_ref, w3_ref, wlin_ref, z_ref):
    d = dinv_ref[:, :1]
    h2 = d * (acc_ref[0] + acc_ref[1] + g2_ref[...]) + b2_ref[...]
    h2 = jnp.maximum(h2, 0.0)
    w3l = jnp.dot(w3_ref[...], wlin_ref[...], preferred_element_type=jnp.float32)
    z = jnp.dot(h2, w3l, preferred_element_type=jnp.float32) * d
    z_ref[...] = jnp.broadcast_to(z, (z.shape[0], 16))


def _head_body(acc_ref, z_ref, dinv_ref, batch_ref, b3_ref, wlin_ref, blin_ref, out_ref):
    c = jnp.dot(b3_ref[...], wlin_ref[...], preferred_element_type=jnp.float32)
    h3 = dinv_ref[:, :1] * (acc_ref[0][:, :1] + acc_ref[1][:, :1] + z_ref[:, :1]) + c
    onehot = (batch_ref[...] == lax.broadcasted_iota(jnp.int32, (1, NG), 1)
              ).astype(jnp.float32)
    sums = jnp.sum(onehot * h3, axis=0)
    counts = jnp.sum(onehot, axis=0)
    out_ref[...] = (sums / jnp.maximum(counts, 1.0))[:, None] + blin_ref[...]


_RB_TC = 2560


def kernel(x, edge_index, batch, W1, b1, W2, b2, W3, b3, Wlin, blin):
    f32 = jnp.float32
    e_used = 16 * (NCH0 + NCH1) * K
    epad0 = N + (jnp.arange(max(0, e_used - E), dtype=jnp.int32) % (NPAD - N))

    def _split(idx):
        flat = jnp.concatenate([idx.astype(jnp.int32), epad0])[:e_used]
        a = flat[:16 * NCH0 * K].reshape(16, NCH0, K)
        b = flat[16 * NCH0 * K:].reshape(16, NCH1, K)
        a = jnp.pad(a, ((0, 0), (0, NCHM - NCH0), (0, 0)), constant_values=PADROW)
        b = jnp.pad(b, ((0, 0), (0, NCHM - NCH1), (0, 0)), constant_values=PADROW)
        return jnp.concatenate([a, b], axis=0)

    src_r = _split(edge_index[0])
    dst_r = _split(edge_index[1])
    x_pad = jnp.pad(x, ((0, NPAD - N), (0, 16 - D_IN)))
    batch_p = jnp.pad(batch.astype(jnp.int32), (0, NPAD - N), constant_values=NG)
    batch_p = batch_p.reshape(NPAD, 1)
    W1p = jnp.pad(W1, ((0, 16 - D_IN), (0, 0)))
    b1r, b2r, b3r = b1.reshape(1, DH), b2.reshape(1, DH), b3.reshape(1, DH)
    blinr = blin.reshape(1, 1)

    deg2 = _deg_kernel(dst_r)

    xs, dinv = pl.pallas_call(
        _scale_body,
        out_shape=[jax.ShapeDtypeStruct((NPAD, 16), f32),
                   jax.ShapeDtypeStruct((NPAD, 16), f32)],
    )(deg2, x_pad)

    acc1 = _spmm16(src_r, dst_r, xs)

    ng = NPAD // _RB_TC
    g2 = pl.pallas_call(
        _layer12_body,
        grid=(ng,),
        in_specs=[
            pl.BlockSpec((2, _RB_TC, 16), lambda i: (0, i, 0)),
            pl.BlockSpec((_RB_TC, 16), lambda i: (i, 0)),
            pl.BlockSpec((_RB_TC, 16), lambda i: (i, 0)),
            pl.BlockSpec((16, DH), lambda i: (0, 0)),
            pl.BlockSpec((1, DH), lambda i: (0, 0)),
            pl.BlockSpec((DH, DH), lambda i: (0, 0)),
        ],
        out_specs=pl.BlockSpec((_RB_TC, DH), lambda i: (i, 0)),
        out_shape=jax.ShapeDtypeStruct((NPAD, DH), f32),
    )(acc1, xs, dinv, W1p, b1r, W2)

    acc2 = _spmm128(src_r, dst_r, g2)

    z = pl.pallas_call(
        _layer23_body,
        grid=(ng,),
        in_specs=[
            pl.BlockSpec((2, _RB_TC, DH), lambda i: (0, i, 0)),
            pl.BlockSpec((_RB_TC, DH), lambda i: (i, 0)),
            pl.BlockSpec((_RB_TC, 16), lambda i: (i, 0)),
            pl.BlockSpec((1, DH), lambda i: (0, 0)),
            pl.BlockSpec((DH, DH), lambda i: (0, 0)),
            pl.BlockSpec((DH, 1), lambda i: (0, 0)),
        ],
        out_specs=pl.BlockSpec((_RB_TC, 16), lambda i: (i, 0)),
        out_shape=jax.ShapeDtypeStruct((NPAD, 16), f32),
    )(acc2, g2, dinv, b2r, W3, Wlin)

    acc3 = _spmm16(src_r, dst_r, z)

    out = pl.pallas_call(
        _head_body,
        out_shape=jax.ShapeDtypeStruct((NG, 1), f32),
    )(acc3, z, dinv, batch_p, b3r, Wlin, blinr)
    return out


# speedup vs baseline: 5.4590x; 1.0573x over previous
import functools

import jax
import jax.numpy as jnp
from jax import lax
from jax.experimental import pallas as pl
from jax.experimental.pallas import tpu as pltpu
from jax.experimental.pallas import tpu_sc as plsc

N = 10000
NG = 64
D_IN = 11
DH = 128
NPAD = 10240
PADROW = NPAD - 1
E = 320000
K = 128
NCH0 = 80
NCH1 = 80
NCHM = max(NCH0, NCH1)
NCH = NCHM
NW = 32
EPAD = NW * NCHM * K
RPT = NPAD // 16
RB = 128

_MESH = plsc.VectorSubcoreMesh(
    core_axis_name="c", subcore_axis_name="s", num_cores=2, num_subcores=16)

def _zero_acc_slice(s, zbuf, acc, F):
    zero16 = jnp.zeros((16,), jnp.float32)
    for i in range(16):
        for k in range(F // 16):
            zbuf[i, pl.ds(16 * k, 16)] = zero16

    @pl.loop(0, RPT // 16)
    def _zero(j):
        pltpu.sync_copy(zbuf, acc.at[pl.ds(s * RPT + j * 16, 16)])


def _readout(c, s, acc, stage, out_hbm):
    @pl.loop(0, RPT // RB)
    def _read(j):
        pltpu.sync_copy(acc.at[pl.ds(s * RPT + j * RB, RB)], stage)
        pltpu.sync_copy(stage, out_hbm.at[c, pl.ds(s * RPT + j * RB, RB)])


IG = 8


def _spmm_body(F, src_hbm, dst_hbm, table_hbm, out_hbm,
               sidx, didx, rows2, zbuf, acc, sem, sem_i):
    c = lax.axis_index("c")
    s = lax.axis_index("s")
    wid = c * 16 + s
    nch = jnp.where(c == 0, NCH0, NCH1)

    _zero_acc_slice(s, zbuf, acc, F)

    pltpu.async_copy(src_hbm.at[wid, pl.ds(0, IG)], sidx.at[0], sem_i)
    pltpu.async_copy(dst_hbm.at[wid, pl.ds(0, IG)], didx.at[0], sem_i)
    plsc.subcore_barrier()

    @pl.loop(0, nch + 1)
    def _edges(j):
        g = j // IG
        h = lax.rem(g, 2)
        jr = lax.rem(j, IG)

        @pl.when(j < nch)
        def _gather():
            @pl.when(jr == 0)
            def _wait_refill():
                pltpu.make_async_copy(
                    src_hbm.at[wid, pl.ds(g * IG, IG)], sidx.at[h], sem_i).wait()
                pltpu.make_async_copy(
                    dst_hbm.at[wid, pl.ds(g * IG, IG)], didx.at[h], sem_i).wait()

            pltpu.async_copy(
                table_hbm.at[sidx.at[h, jr]], rows2.at[lax.rem(j, 2)], sem)

        @pl.when(j >= 1)
        def _scatter():
            jm = j - 1
            hm = lax.rem(jm // IG, 2)
            b = lax.rem(jm, 2)
            pltpu.make_async_copy(
                table_hbm.at[sidx.at[hm, lax.rem(jm, IG)]], rows2.at[b], sem).wait()
            pltpu.sync_copy(
                rows2.at[b], acc.at[didx.at[hm, lax.rem(jm, IG)]], add=True)

        @pl.when((jr == 0) & (j + IG < nch))
        def _refill():
            h2 = lax.rem(g + 1, 2)
            pltpu.async_copy(src_hbm.at[wid, pl.ds((g + 1) * IG, IG)], sidx.at[h2], sem_i)
            pltpu.async_copy(dst_hbm.at[wid, pl.ds((g + 1) * IG, IG)], didx.at[h2], sem_i)

    plsc.subcore_barrier()
    _readout(c, s, acc, rows2.at[0], out_hbm)


D16 = 4


def _spmm16_body(src_hbm, dst_hbm, table_hbm, out_hbm,
                 sidx, didx, rows, zbuf, acc, sem_g, sem_s):
    c = lax.axis_index("c")
    s = lax.axis_index("s")
    wid = c * 16 + s

    _zero_acc_slice(s, zbuf, acc, 16)

    pltpu.sync_copy(src_hbm.at[wid], sidx)
    pltpu.sync_copy(dst_hbm.at[wid], didx)
    plsc.subcore_barrier()

    for jj in range(D16 - 1):
        pltpu.async_copy(table_hbm.at[sidx.at[jj]], rows.at[jj], sem_g)

    @pl.loop(0, NCHM)
    def _edges(j):
        b = lax.rem(j, D16)
        pltpu.make_async_copy(table_hbm.at[sidx.at[j]], rows.at[b], sem_g).wait()
        pltpu.async_copy(rows.at[b], acc.at[didx.at[j]], sem_s, add=True)

        @pl.when(j >= 1)
        def _wait_scatter():
            jm = j - 1
            pltpu.make_async_copy(
                rows.at[lax.rem(jm, D16)], acc.at[didx.at[jm]], sem_s).wait()

        @pl.when(j + (D16 - 1) < NCHM)
        def _next_gather():
            jn = j + D16 - 1
            pltpu.async_copy(table_hbm.at[sidx.at[jn]], rows.at[lax.rem(jn, D16)], sem_g)

    pltpu.make_async_copy(
        rows.at[lax.rem(NCHM - 1, D16)], acc.at[didx.at[NCHM - 1]], sem_s).wait()
    plsc.subcore_barrier()
    _readout(c, s, acc, rows.at[0], out_hbm)


def _deg_body(dst_hbm, out_hbm, didx, rows, zbuf, acc, sem):
    c = lax.axis_index("c")
    s = lax.axis_index("s")
    wid = c * 16 + s

    _zero_acc_slice(s, zbuf, acc, 16)

    one16 = jnp.ones((16,), jnp.float32)
    for i in range(RB):
        rows[i, pl.ds(0, 16)] = one16

    pltpu.sync_copy(dst_hbm.at[wid], didx)
    plsc.subcore_barrier()

    @pl.loop(0, NCHM)
    def _edges(j):
        pltpu.async_copy(rows, acc.at[didx.at[j]], sem, add=True)

        @pl.when(j >= 1)
        def _wait_prev():
            pltpu.make_async_copy(rows, acc.at[didx.at[j - 1]], sem).wait()

    pltpu.make_async_copy(rows, acc.at[didx.at[NCHM - 1]], sem).wait()
    plsc.subcore_barrier()
    _readout(c, s, acc, rows, out_hbm)


def _make_spmm(F, tc_tiling=True):
    scratch = [
        pltpu.VMEM((2, IG, K), jnp.int32),
        pltpu.VMEM((2, IG, K), jnp.int32),
        pltpu.VMEM((2, RB, F), jnp.float32),
        pltpu.VMEM((16, F), jnp.float32),
        pltpu.VMEM_SHARED((NPAD, F), jnp.float32),
        pltpu.SemaphoreType.DMA,
        pltpu.SemaphoreType.DMA,
    ]
    return pl.kernel(
        functools.partial(_spmm_body, F),
        out_type=jax.ShapeDtypeStruct((2, NPAD, F), jnp.float32),
        mesh=_MESH,
        scratch_types=scratch,
        compiler_params=pltpu.CompilerParams(use_tc_tiling_on_sc=tc_tiling),
        name=f"sc_spmm_f{F}",
    )


_DEG_SCRATCH = [
    pltpu.VMEM((NCHM, K), jnp.int32),
    pltpu.VMEM((RB, 16), jnp.float32),
    pltpu.VMEM((16, 16), jnp.float32),
    pltpu.VMEM_SHARED((NPAD, 16), jnp.float32),
    pltpu.SemaphoreType.DMA,
]

_deg_kernel = pl.kernel(
    _deg_body,
    out_type=jax.ShapeDtypeStruct((2, NPAD, 16), jnp.float32),
    mesh=_MESH,
    scratch_types=_DEG_SCRATCH,
    name="sc_degree",
)

_spmm128 = _make_spmm(128)
_SPMM16_SCRATCH = [
    pltpu.VMEM((NCHM, K), jnp.int32),
    pltpu.VMEM((NCHM, K), jnp.int32),
    pltpu.VMEM((D16, RB, 16), jnp.float32),
    pltpu.VMEM((16, 16), jnp.float32),
    pltpu.VMEM_SHARED((NPAD, 16), jnp.float32),
    pltpu.SemaphoreType.DMA,
    pltpu.SemaphoreType.DMA,
]

_spmm16 = pl.kernel(
    _spmm16_body,
    out_type=jax.ShapeDtypeStruct((2, NPAD, 16), jnp.float32),
    mesh=_MESH,
    scratch_types=_SPMM16_SCRATCH,
    compiler_params=pltpu.CompilerParams(use_tc_tiling_on_sc=False),
    name="sc_spmm_f16",
)


def _scale_body(deg_ref, x_ref, xs_ref, dinv_ref):
    deg = deg_ref[0] + deg_ref[1] + 1.0
    dinv = lax.rsqrt(deg)
    rowid = lax.broadcasted_iota(jnp.int32, (NPAD, 16), 0)
    dinv = jnp.where(rowid < N, dinv, 0.0)
    xs_ref[...] = x_ref[...] * dinv
    dinv_ref[...] = dinv


def _layer12_body(acc_ref, xs_ref, dinv_ref, w1_ref, b1_ref, w2_ref, g2_ref):
    pre = dinv_ref[...] * (acc_ref[0] + acc_ref[1] + xs_ref[...])
    h1 = jnp.dot(pre, w1_ref[...], preferred_element_type=jnp.float32) + b1_ref[...]
    h1 = jnp.maximum(h1, 0.0)
    g2 = jnp.dot(h1, w2_ref[...], preferred_element_type=jnp.float32)
    g2_ref[...] = g2 * dinv_ref[:, :1]


def _layer23_body(acc_ref, g2_ref, dinv_ref, b2_ref, w3_ref, wlin_ref, z_ref):
    d = dinv_ref[:, :1]
    h2 = d * (acc_ref[0] + acc_ref[1] + g2_ref[...]) + b2_ref[...]
    h2 = jnp.maximum(h2, 0.0)
    w3l = jnp.dot(w3_ref[...], wlin_ref[...], preferred_element_type=jnp.float32)
    z = jnp.dot(h2, w3l, preferred_element_type=jnp.float32) * d
    z_ref[...] = jnp.broadcast_to(z, (z.shape[0], 16))


def _head_body(acc_ref, z_ref, dinv_ref, batch_ref, b3_ref, wlin_ref, blin_ref, out_ref):
    c = jnp.dot(b3_ref[...], wlin_ref[...], preferred_element_type=jnp.float32)
    h3 = dinv_ref[:, :1] * (acc_ref[0][:, :1] + acc_ref[1][:, :1] + z_ref[:, :1]) + c
    onehot = (batch_ref[...] == lax.broadcasted_iota(jnp.int32, (1, NG), 1)
              ).astype(jnp.float32)
    sums = jnp.sum(onehot * h3, axis=0)
    counts = jnp.sum(onehot, axis=0)
    out_ref[...] = (sums / jnp.maximum(counts, 1.0))[:, None] + blin_ref[...]


_RB_TC = 2560


def kernel(x, edge_index, batch, W1, b1, W2, b2, W3, b3, Wlin, blin):
    f32 = jnp.float32
    e_used = 16 * (NCH0 + NCH1) * K
    epad0 = N + (jnp.arange(max(0, e_used - E), dtype=jnp.int32) % (NPAD - N))

    def _split(idx):
        flat = jnp.concatenate([idx.astype(jnp.int32), epad0])[:e_used]
        a = flat[:16 * NCH0 * K].reshape(16, NCH0, K)
        b = flat[16 * NCH0 * K:].reshape(16, NCH1, K)
        a = jnp.pad(a, ((0, 0), (0, NCHM - NCH0), (0, 0)), constant_values=PADROW)
        b = jnp.pad(b, ((0, 0), (0, NCHM - NCH1), (0, 0)), constant_values=PADROW)
        return jnp.concatenate([a, b], axis=0)

    src_r = _split(edge_index[0])
    dst_r = _split(edge_index[1])
    x_pad = jnp.pad(x, ((0, NPAD - N), (0, 16 - D_IN)))
    batch_p = jnp.pad(batch.astype(jnp.int32), (0, NPAD - N), constant_values=NG)
    batch_p = batch_p.reshape(NPAD, 1)
    W1p = jnp.pad(W1, ((0, 16 - D_IN), (0, 0)))
    b1r, b2r, b3r = b1.reshape(1, DH), b2.reshape(1, DH), b3.reshape(1, DH)
    blinr = blin.reshape(1, 1)

    deg2 = _deg_kernel(dst_r)

    xs, dinv = pl.pallas_call(
        _scale_body,
        out_shape=[jax.ShapeDtypeStruct((NPAD, 16), f32),
                   jax.ShapeDtypeStruct((NPAD, 16), f32)],
    )(deg2, x_pad)

    acc1 = _spmm16(src_r, dst_r, xs)

    ng = NPAD // _RB_TC
    g2 = pl.pallas_call(
        _layer12_body,
        grid=(ng,),
        in_specs=[
            pl.BlockSpec((2, _RB_TC, 16), lambda i: (0, i, 0)),
            pl.BlockSpec((_RB_TC, 16), lambda i: (i, 0)),
            pl.BlockSpec((_RB_TC, 16), lambda i: (i, 0)),
            pl.BlockSpec((16, DH), lambda i: (0, 0)),
            pl.BlockSpec((1, DH), lambda i: (0, 0)),
            pl.BlockSpec((DH, DH), lambda i: (0, 0)),
        ],
        out_specs=pl.BlockSpec((_RB_TC, DH), lambda i: (i, 0)),
        out_shape=jax.ShapeDtypeStruct((NPAD, DH), f32),
    )(acc1, xs, dinv, W1p, b1r, W2)

    acc2 = _spmm128(src_r, dst_r, g2)

    z = pl.pallas_call(
        _layer23_body,
        grid=(ng,),
        in_specs=[
            pl.BlockSpec((2, _RB_TC, DH), lambda i: (0, i, 0)),
            pl.BlockSpec((_RB_TC, DH), lambda i: (i, 0)),
            pl.BlockSpec((_RB_TC, 16), lambda i: (i, 0)),
            pl.BlockSpec((1, DH), lambda i: (0, 0)),
            pl.BlockSpec((DH, DH), lambda i: (0, 0)),
            pl.BlockSpec((DH, 1), lambda i: (0, 0)),
        ],
        out_specs=pl.BlockSpec((_RB_TC, 16), lambda i: (i, 0)),
        out_shape=jax.ShapeDtypeStruct((NPAD, 16), f32),
    )(acc2, g2, dinv, b2r, W3, Wlin)

    acc3 = _spmm16(src_r, dst_r, z)

    out = pl.pallas_call(
        _head_body,
        out_shape=jax.ShapeDtypeStruct((NG, 1), f32),
    )(acc3, z, dinv, batch_p, b3r, Wlin, blinr)
    return out


# speedup vs baseline: 6.0259x; 1.1039x over previous
import functools

import jax
import jax.numpy as jnp
from jax import lax
from jax.experimental import pallas as pl
from jax.experimental.pallas import tpu as pltpu
from jax.experimental.pallas import tpu_sc as plsc

N = 10000
NG = 64
D_IN = 11
DH = 128
NPAD = 10240
PADROW = NPAD - 1
E = 320000
K = 128
NCH0 = 80
NCH1 = 80
NCHM = max(NCH0, NCH1)
NCH = NCHM
NW = 32
EPAD = NW * NCHM * K
RPT = NPAD // 16
RB = 128

_MESH = plsc.VectorSubcoreMesh(
    core_axis_name="c", subcore_axis_name="s", num_cores=2, num_subcores=16)

def _zero_acc_slice(s, zbuf, acc, F):
    zero16 = jnp.zeros((16,), jnp.float32)
    for i in range(16):
        for k in range(F // 16):
            zbuf[i, pl.ds(16 * k, 16)] = zero16

    @pl.loop(0, RPT // 16)
    def _zero(j):
        pltpu.sync_copy(zbuf, acc.at[pl.ds(s * RPT + j * 16, 16)])


def _readout(c, s, acc, stage, out_hbm):
    @pl.loop(0, RPT // RB)
    def _read(j):
        pltpu.sync_copy(acc.at[pl.ds(s * RPT + j * RB, RB)], stage)
        pltpu.sync_copy(stage, out_hbm.at[c, pl.ds(s * RPT + j * RB, RB)])


IG = 8


def _spmm_body(F, src_hbm, dst_hbm, table_hbm, out_hbm,
               sidx, didx, rows2, zbuf, acc, sem, sem_i):
    c = lax.axis_index("c")
    s = lax.axis_index("s")
    wid = c * 16 + s
    nch = jnp.where(c == 0, NCH0, NCH1)

    _zero_acc_slice(s, zbuf, acc, F)

    pltpu.async_copy(src_hbm.at[wid, pl.ds(0, IG)], sidx.at[0], sem_i)
    pltpu.async_copy(dst_hbm.at[wid, pl.ds(0, IG)], didx.at[0], sem_i)
    plsc.subcore_barrier()

    @pl.loop(0, nch + 1)
    def _edges(j):
        g = j // IG
        h = lax.rem(g, 2)
        jr = lax.rem(j, IG)

        @pl.when(j < nch)
        def _gather():
            @pl.when(jr == 0)
            def _wait_refill():
                pltpu.make_async_copy(
                    src_hbm.at[wid, pl.ds(g * IG, IG)], sidx.at[h], sem_i).wait()
                pltpu.make_async_copy(
                    dst_hbm.at[wid, pl.ds(g * IG, IG)], didx.at[h], sem_i).wait()

            pltpu.async_copy(
                table_hbm.at[sidx.at[h, jr]], rows2.at[lax.rem(j, 2)], sem)

        @pl.when(j >= 1)
        def _scatter():
            jm = j - 1
            hm = lax.rem(jm // IG, 2)
            b = lax.rem(jm, 2)
            pltpu.make_async_copy(
                table_hbm.at[sidx.at[hm, lax.rem(jm, IG)]], rows2.at[b], sem).wait()
            pltpu.sync_copy(
                rows2.at[b], acc.at[didx.at[hm, lax.rem(jm, IG)]], add=True)

        @pl.when((jr == 0) & (j + IG < nch))
        def _refill():
            h2 = lax.rem(g + 1, 2)
            pltpu.async_copy(src_hbm.at[wid, pl.ds((g + 1) * IG, IG)], sidx.at[h2], sem_i)
            pltpu.async_copy(dst_hbm.at[wid, pl.ds((g + 1) * IG, IG)], didx.at[h2], sem_i)

    plsc.subcore_barrier()
    _readout(c, s, acc, rows2.at[0], out_hbm)


D16 = 8


def _spmm16_body(src_hbm, dst_hbm, table_hbm, out_hbm,
                 sidx, didx, rows, zbuf, acc, sem_g, sem_s):
    c = lax.axis_index("c")
    s = lax.axis_index("s")
    wid = c * 16 + s

    _zero_acc_slice(s, zbuf, acc, 16)

    pltpu.sync_copy(src_hbm.at[wid], sidx)
    pltpu.sync_copy(dst_hbm.at[wid], didx)
    plsc.subcore_barrier()

    for jj in range(D16 - 1):
        pltpu.async_copy(table_hbm.at[sidx.at[jj]], rows.at[jj], sem_g)

    @pl.loop(0, NCHM)
    def _edges(j):
        b = lax.rem(j, D16)
        pltpu.make_async_copy(table_hbm.at[sidx.at[j]], rows.at[b], sem_g).wait()
        pltpu.async_copy(rows.at[b], acc.at[didx.at[j]], sem_s, add=True)

        @pl.when(j >= 1)
        def _wait_scatter():
            jm = j - 1
            pltpu.make_async_copy(
                rows.at[lax.rem(jm, D16)], acc.at[didx.at[jm]], sem_s).wait()

        @pl.when(j + (D16 - 1) < NCHM)
        def _next_gather():
            jn = j + D16 - 1
            pltpu.async_copy(table_hbm.at[sidx.at[jn]], rows.at[lax.rem(jn, D16)], sem_g)

    pltpu.make_async_copy(
        rows.at[lax.rem(NCHM - 1, D16)], acc.at[didx.at[NCHM - 1]], sem_s).wait()
    plsc.subcore_barrier()
    _readout(c, s, acc, rows.at[0], out_hbm)


def _deg_body(dst_hbm, out_hbm, didx, rows, zbuf, acc, sem):
    c = lax.axis_index("c")
    s = lax.axis_index("s")
    wid = c * 16 + s

    _zero_acc_slice(s, zbuf, acc, 16)

    one16 = jnp.ones((16,), jnp.float32)
    for i in range(RB):
        rows[i, pl.ds(0, 16)] = one16

    pltpu.sync_copy(dst_hbm.at[wid], didx)
    plsc.subcore_barrier()

    @pl.loop(0, NCHM)
    def _edges(j):
        pltpu.async_copy(rows, acc.at[didx.at[j]], sem, add=True)

        @pl.when(j >= 1)
        def _wait_prev():
            pltpu.make_async_copy(rows, acc.at[didx.at[j - 1]], sem).wait()

    pltpu.make_async_copy(rows, acc.at[didx.at[NCHM - 1]], sem).wait()
    plsc.subcore_barrier()
    _readout(c, s, acc, rows, out_hbm)


def _make_spmm(F, tc_tiling=True):
    scratch = [
        pltpu.VMEM((2, IG, K), jnp.int32),
        pltpu.VMEM((2, IG, K), jnp.int32),
        pltpu.VMEM((2, RB, F), jnp.float32),
        pltpu.VMEM((16, F), jnp.float32),
        pltpu.VMEM_SHARED((NPAD, F), jnp.float32),
        pltpu.SemaphoreType.DMA,
        pltpu.SemaphoreType.DMA,
    ]
    return pl.kernel(
        functools.partial(_spmm_body, F),
        out_type=jax.ShapeDtypeStruct((2, NPAD, F), jnp.float32),
        mesh=_MESH,
        scratch_types=scratch,
        compiler_params=pltpu.CompilerParams(use_tc_tiling_on_sc=tc_tiling),
        name=f"sc_spmm_f{F}",
    )


_DEG_SCRATCH = [
    pltpu.VMEM((NCHM, K), jnp.int32),
    pltpu.VMEM((RB, 16), jnp.float32),
    pltpu.VMEM((16, 16), jnp.float32),
    pltpu.VMEM_SHARED((NPAD, 16), jnp.float32),
    pltpu.SemaphoreType.DMA,
]

_deg_kernel = pl.kernel(
    _deg_body,
    out_type=jax.ShapeDtypeStruct((2, NPAD, 16), jnp.float32),
    mesh=_MESH,
    scratch_types=_DEG_SCRATCH,
    name="sc_degree",
)

_spmm128 = _make_spmm(128)
_SPMM16_SCRATCH = [
    pltpu.VMEM((NCHM, K), jnp.int32),
    pltpu.VMEM((NCHM, K), jnp.int32),
    pltpu.VMEM((D16, RB, 16), jnp.float32),
    pltpu.VMEM((16, 16), jnp.float32),
    pltpu.VMEM_SHARED((NPAD, 16), jnp.float32),
    pltpu.SemaphoreType.DMA,
    pltpu.SemaphoreType.DMA,
]

_spmm16 = pl.kernel(
    _spmm16_body,
    out_type=jax.ShapeDtypeStruct((2, NPAD, 16), jnp.float32),
    mesh=_MESH,
    scratch_types=_SPMM16_SCRATCH,
    compiler_params=pltpu.CompilerParams(use_tc_tiling_on_sc=False),
    name="sc_spmm_f16",
)


def _scale_body(deg_ref, x_ref, xs_ref, dinv_ref):
    deg = deg_ref[0] + deg_ref[1] + 1.0
    dinv = lax.rsqrt(deg)
    rowid = lax.broadcasted_iota(jnp.int32, (NPAD, 16), 0)
    dinv = jnp.where(rowid < N, dinv, 0.0)
    xs_ref[...] = x_ref[...] * dinv
    dinv_ref[...] = dinv


def _layer12_body(acc_ref, xs_ref, dinv_ref, w1_ref, b1_ref, w2_ref, g2_ref):
    pre = dinv_ref[...] * (acc_ref[0] + acc_ref[1] + xs_ref[...])
    h1 = jnp.dot(pre, w1_ref[...], preferred_element_type=jnp.float32) + b1_ref[...]
    h1 = jnp.maximum(h1, 0.0)
    g2 = jnp.dot(h1, w2_ref[...], preferred_element_type=jnp.float32)
    g2_ref[...] = g2 * dinv_ref[:, :1]


def _layer23_body(acc_ref, g2_ref, dinv_ref, b2_ref, w3_ref, wlin_ref, z_ref):
    d = dinv_ref[:, :1]
    h2 = d * (acc_ref[0] + acc_ref[1] + g2_ref[...]) + b2_ref[...]
    h2 = jnp.maximum(h2, 0.0)
    w3l = jnp.dot(w3_ref[...], wlin_ref[...], preferred_element_type=jnp.float32)
    z = jnp.dot(h2, w3l, preferred_element_type=jnp.float32) * d
    z_ref[...] = jnp.broadcast_to(z, (z.shape[0], 16))


def _head_body(acc_ref, z_ref, dinv_ref, batch_ref, b3_ref, wlin_ref, blin_ref, out_ref):
    c = jnp.dot(b3_ref[...], wlin_ref[...], preferred_element_type=jnp.float32)
    h3 = dinv_ref[:, :1] * (acc_ref[0][:, :1] + acc_ref[1][:, :1] + z_ref[:, :1]) + c
    onehot = (batch_ref[...] == lax.broadcasted_iota(jnp.int32, (1, NG), 1)
              ).astype(jnp.float32)
    sums = jnp.sum(onehot * h3, axis=0)
    counts = jnp.sum(onehot, axis=0)
    out_ref[...] = (sums / jnp.maximum(counts, 1.0))[:, None] + blin_ref[...]


_RB_TC = 2560


def kernel(x, edge_index, batch, W1, b1, W2, b2, W3, b3, Wlin, blin):
    f32 = jnp.float32
    e_used = 16 * (NCH0 + NCH1) * K
    epad0 = N + (jnp.arange(max(0, e_used - E), dtype=jnp.int32) % (NPAD - N))

    def _split(idx):
        flat = jnp.concatenate([idx.astype(jnp.int32), epad0])[:e_used]
        a = flat[:16 * NCH0 * K].reshape(16, NCH0, K)
        b = flat[16 * NCH0 * K:].reshape(16, NCH1, K)
        a = jnp.pad(a, ((0, 0), (0, NCHM - NCH0), (0, 0)), constant_values=PADROW)
        b = jnp.pad(b, ((0, 0), (0, NCHM - NCH1), (0, 0)), constant_values=PADROW)
        return jnp.concatenate([a, b], axis=0)

    src_r = _split(edge_index[0])
    dst_r = _split(edge_index[1])
    x_pad = jnp.pad(x, ((0, NPAD - N), (0, 16 - D_IN)))
    batch_p = jnp.pad(batch.astype(jnp.int32), (0, NPAD - N), constant_values=NG)
    batch_p = batch_p.reshape(NPAD, 1)
    W1p = jnp.pad(W1, ((0, 16 - D_IN), (0, 0)))
    b1r, b2r, b3r = b1.reshape(1, DH), b2.reshape(1, DH), b3.reshape(1, DH)
    blinr = blin.reshape(1, 1)

    deg2 = _deg_kernel(dst_r)

    xs, dinv = pl.pallas_call(
        _scale_body,
        out_shape=[jax.ShapeDtypeStruct((NPAD, 16), f32),
                   jax.ShapeDtypeStruct((NPAD, 16), f32)],
    )(deg2, x_pad)

    acc1 = _spmm16(src_r, dst_r, xs)

    ng = NPAD // _RB_TC
    g2 = pl.pallas_call(
        _layer12_body,
        grid=(ng,),
        in_specs=[
            pl.BlockSpec((2, _RB_TC, 16), lambda i: (0, i, 0)),
            pl.BlockSpec((_RB_TC, 16), lambda i: (i, 0)),
            pl.BlockSpec((_RB_TC, 16), lambda i: (i, 0)),
            pl.BlockSpec((16, DH), lambda i: (0, 0)),
            pl.BlockSpec((1, DH), lambda i: (0, 0)),
            pl.BlockSpec((DH, DH), lambda i: (0, 0)),
        ],
        out_specs=pl.BlockSpec((_RB_TC, DH), lambda i: (i, 0)),
        out_shape=jax.ShapeDtypeStruct((NPAD, DH), f32),
    )(acc1, xs, dinv, W1p, b1r, W2)

    acc2 = _spmm128(src_r, dst_r, g2)

    z = pl.pallas_call(
        _layer23_body,
        grid=(ng,),
        in_specs=[
            pl.BlockSpec((2, _RB_TC, DH), lambda i: (0, i, 0)),
            pl.BlockSpec((_RB_TC, DH), lambda i: (i, 0)),
            pl.BlockSpec((_RB_TC, 16), lambda i: (i, 0)),
            pl.BlockSpec((1, DH), lambda i: (0, 0)),
            pl.BlockSpec((DH, DH), lambda i: (0, 0)),
            pl.BlockSpec((DH, 1), lambda i: (0, 0)),
        ],
        out_specs=pl.BlockSpec((_RB_TC, 16), lambda i: (i, 0)),
        out_shape=jax.ShapeDtypeStruct((NPAD, 16), f32),
    )(acc2, g2, dinv, b2r, W3, Wlin)

    acc3 = _spmm16(src_r, dst_r, z)

    out = pl.pallas_call(
        _head_body,
        out_shape=jax.ShapeDtypeStruct((NG, 1), f32),
    )(acc3, z, dinv, batch_p, b3r, Wlin, blinr)
    return out
